# Initial kernel scaffold; baseline (speedup 1.0000x reference)
import jax
import jax.numpy as jnp
from jax.experimental import pallas as pl


def kernel(cross_features, emb_table, W_ih, W_hh, b_ih, b_hh, paths):
    raise NotImplementedError("write your pallas kernel here")


# speedup vs baseline: 1.6373x; 1.6373x over previous
import functools

import jax
import jax.numpy as jnp
from jax import lax
from jax.experimental import pallas as pl
from jax.experimental.pallas import tpu as pltpu
from jax.experimental.pallas import tpu_sc as plsc

N_TREES = 8
LEAVES_PER_TREE = 16
N_LEAVES = N_TREES * LEAVES_PER_TREE
PATH_LEN = 8
EMBED_DIM = 128
BATCH = 256
N_ROWS = N_LEAVES * PATH_LEN


@functools.cache
def _make_sc_gather():
    info = plsc.get_sparse_core_info()
    nw = info.num_cores * info.num_subcores
    rows_per_w = N_ROWS // nw
    mesh = plsc.VectorSubcoreMesh(core_axis_name="c", subcore_axis_name="s")

    @functools.partial(
        pl.kernel,
        mesh=mesh,
        out_type=jax.ShapeDtypeStruct((N_ROWS, EMBED_DIM), jnp.float32),
        scratch_types=[
            pltpu.VMEM((rows_per_w,), jnp.int32),
            pltpu.VMEM((rows_per_w, EMBED_DIM), jnp.float32),
            pltpu.SemaphoreType.DMA,
        ],
    )
    def gather_kernel(table_hbm, idx_hbm, out_hbm, idx_v, rows_v, sem):
        wid = lax.axis_index("s") * info.num_cores + lax.axis_index("c")
        base = wid * rows_per_w
        pltpu.sync_copy(idx_hbm.at[pl.ds(base, rows_per_w)], idx_v)
        pltpu.async_copy(table_hbm.at[idx_v], rows_v, sem).wait()
        pltpu.sync_copy(rows_v, out_hbm.at[pl.ds(base, rows_per_w)])

    return gather_kernel


def _tc_body(pe_ref, cf_ref, wi_ref, wh_ref, bi_ref, bh_ref, out_ref):
    h = jnp.zeros((N_LEAVES, EMBED_DIM), dtype=jnp.float32)
    c = jnp.zeros((N_LEAVES, EMBED_DIM), dtype=jnp.float32)
    bias = bi_ref[...] + bh_ref[...]
    wi = wi_ref[...]
    wh = wh_ref[...]
    H = EMBED_DIM
    for t in range(PATH_LEN):
        x = pe_ref[t * N_LEAVES:(t + 1) * N_LEAVES, :]
        gates = (
            jnp.dot(x, wi, preferred_element_type=jnp.float32)
            + jnp.dot(h, wh, preferred_element_type=jnp.float32)
            + bias
        )
        gi = gates[:, 0:H]
        gf = gates[:, H:2 * H]
        gg = gates[:, 2 * H:3 * H]
        go = gates[:, 3 * H:4 * H]
        si = 1.0 / (1.0 + jnp.exp(-gi))
        sf = 1.0 / (1.0 + jnp.exp(-gf))
        so = 1.0 / (1.0 + jnp.exp(-go))
        c = sf * c + si * jnp.tanh(gg)
        h = so * jnp.tanh(c)

    cf = cf_ref[...]
    lane = lax.broadcasted_iota(jnp.int32, (BATCH, N_LEAVES), 1)
    jl = lane & (LEAVES_PER_TREE - 1)
    active = cf > 0.0
    pow2 = lax.bitcast_convert_type((jl + 127) << 23, jnp.float32)
    val = jnp.where(active, pow2, 0.0)
    gi_r = lax.broadcasted_iota(jnp.int32, (N_LEAVES, N_LEAVES), 0) >> 4
    gj_r = lax.broadcasted_iota(jnp.int32, (N_LEAVES, N_LEAVES), 1) >> 4
    blockones = jnp.where(gi_r == gj_r, 1.0, 0.0).astype(jnp.float32)
    valsum = jnp.dot(val, blockones, preferred_element_type=jnp.float32)
    sel = (lax.bitcast_convert_type(valsum, jnp.int32) >> 23) - 127
    onehot = jnp.where(active & (jl == sel) & (valsum > 0.0), 1.0, 0.0)
    tree_id = lane >> 4
    for t in range(N_TREES):
        oh_t = jnp.where(tree_id == t, onehot, 0.0)
        out_ref[:, t, :] = jnp.dot(oh_t, h, preferred_element_type=jnp.float32)


def kernel(cross_features, emb_table, W_ih, W_hh, b_ih, b_hh, paths):
    idx = jnp.transpose(paths).reshape(-1)
    path_emb = _make_sc_gather()(emb_table, idx)
    out = pl.pallas_call(
        _tc_body,
        out_shape=jax.ShapeDtypeStruct((BATCH, N_TREES, EMBED_DIM), jnp.float32),
    )(
        path_emb,
        cross_features,
        jnp.transpose(W_ih),
        jnp.transpose(W_hh),
        b_ih.reshape(1, -1),
        b_hh.reshape(1, -1),
    )
    return out
